# baseline (speedup 1.0000x reference)
import functools
import jax
import jax.numpy as jnp
from jax import lax
from jax.experimental import pallas as pl
from jax.experimental.pallas import tpu as pltpu
from jax.experimental.pallas import tpu_sc as plsc

N = 10000
D = 128
G = 16
E = 320000

NC = 2
NS = 16
NW = NC * NS
K = 128
CH = 80
CH2 = CH // 2
CH0 = 44
CH1 = 116
CHM2 = max(CH0, CH1) // 2
EPW = K * CH
EP = EPW * NW
NP = 10240
RPT = NP // NS

_mesh = plsc.VectorSubcoreMesh(core_axis_name="c", subcore_axis_name="s")


@functools.partial(
    pl.kernel,
    out_type=jax.ShapeDtypeStruct((NC * NP, D), jnp.float32),
    mesh=_mesh,
    scratch_types=[
        pltpu.VMEM((K,), jnp.int32),
        pltpu.VMEM((K, D), jnp.float32),
        pltpu.VMEM_SHARED((NP, D), jnp.float32),
    ],
)
def _deg_kernel(dst_hbm, out_hbm, dst_v, ones_v, acc_sh):
    c = lax.axis_index("c")
    s = lax.axis_index("s")
    wid = s * NC + c

    zero = jnp.zeros((16,), jnp.float32)

    def fill_zero_row(i, _):
        def fill_lane(j, _):
            ones_v[i, pl.ds(j * 16, 16)] = zero
            return 0
        lax.fori_loop(0, D // 16, fill_lane, 0)
        return 0
    lax.fori_loop(0, K, fill_zero_row, 0)

    def zero_acc(i, _):
        pltpu.sync_copy(ones_v, acc_sh.at[pl.ds(s * RPT + i * K, K)])
        return 0
    lax.fori_loop(0, RPT // K, zero_acc, 0)

    one = jnp.full((16,), 1.0, jnp.float32)

    def fill_one_row(i, _):
        def fill_lane(j, _):
            ones_v[i, pl.ds(j * 16, 16)] = one
            return 0
        lax.fori_loop(0, D // 16, fill_lane, 0)
        return 0
    lax.fori_loop(0, K, fill_one_row, 0)
    plsc.subcore_barrier()

    def body(i, _):
        base = pl.multiple_of(wid * EPW + i * K, 8)
        pltpu.sync_copy(dst_hbm.at[pl.ds(base, K)], dst_v)
        pltpu.sync_copy(ones_v, acc_sh.at[dst_v], add=True)
        return 0
    lax.fori_loop(0, CH, body, 0)

    plsc.subcore_barrier()

    def copy_out(i, _):
        pltpu.sync_copy(acc_sh.at[pl.ds(s * RPT + i * K, K)], ones_v)
        pltpu.sync_copy(ones_v, out_hbm.at[pl.ds(c * NP + s * RPT + i * K, K)])
        return 0
    lax.fori_loop(0, RPT // K, copy_out, 0)


@functools.partial(
    pl.kernel,
    out_type=jax.ShapeDtypeStruct((NC * NP, D), jnp.float32),
    mesh=_mesh,
    scratch_types=[
        pltpu.VMEM((K,), jnp.int32),
        pltpu.VMEM((K,), jnp.int32),
        pltpu.VMEM((K,), jnp.int32),
        pltpu.VMEM((K,), jnp.int32),
        pltpu.VMEM((K, D), jnp.float32),
        pltpu.VMEM((K, D), jnp.float32),
        pltpu.VMEM_SHARED((NP, D), jnp.float32),
        pltpu.SemaphoreType.DMA,
        pltpu.SemaphoreType.DMA,
    ],
)
def _agg_kernel(hp_hbm, src_hbm, dst_hbm, out_hbm,
                src_a, dst_a, src_b, dst_b, rows_a, rows_b, acc_sh, ga, gb):
    c = lax.axis_index("c")
    s = lax.axis_index("s")
    zero = jnp.zeros((16,), jnp.float32)

    def fill_zero_row(i, _):
        def fill_lane(j, _):
            rows_a[i, pl.ds(j * 16, 16)] = zero
            return 0
        lax.fori_loop(0, D // 16, fill_lane, 0)
        return 0
    lax.fori_loop(0, K, fill_zero_row, 0)

    def zero_acc(i, _):
        pltpu.sync_copy(rows_a, acc_sh.at[pl.ds(s * RPT + i * K, K)])
        return 0
    lax.fori_loop(0, RPT // K, zero_acc, 0)
    plsc.subcore_barrier()

    n2 = jnp.where(c == 0, CH0 // 2, CH1 // 2)
    cbase = jnp.where(c == 0, s * CH0, 16 * CH0 + s * CH1)

    pltpu.sync_copy(src_hbm.at[pl.ds(cbase * K, K)], src_a)
    pltpu.sync_copy(dst_hbm.at[pl.ds(cbase * K, K)], dst_a)
    pltpu.async_copy(hp_hbm.at[src_a], rows_a, ga)

    def body(j, _):
        @pl.when(j < n2)
        def _():
            i2 = pl.multiple_of((cbase + 2 * j + 1) * K, 8)
            pltpu.sync_copy(src_hbm.at[pl.ds(i2, K)], src_b)
            pltpu.sync_copy(dst_hbm.at[pl.ds(i2, K)], dst_b)
            pltpu.async_copy(hp_hbm.at[src_b], rows_b, gb)
            pltpu.make_async_copy(hp_hbm.at[src_a], rows_a, ga).wait()
            pltpu.sync_copy(rows_a, acc_sh.at[dst_a], add=True)

            @pl.when(j < n2 - 1)
            def _():
                i3 = pl.multiple_of((cbase + 2 * j + 2) * K, 8)
                pltpu.sync_copy(src_hbm.at[pl.ds(i3, K)], src_a)
                pltpu.sync_copy(dst_hbm.at[pl.ds(i3, K)], dst_a)
                pltpu.async_copy(hp_hbm.at[src_a], rows_a, ga)
            pltpu.make_async_copy(hp_hbm.at[src_b], rows_b, gb).wait()
            pltpu.sync_copy(rows_b, acc_sh.at[dst_b], add=True)
        return 0
    lax.fori_loop(0, CHM2, body, 0)

    plsc.subcore_barrier()

    def copy_out(i, _):
        pltpu.sync_copy(acc_sh.at[pl.ds(s * RPT + i * K, K)], rows_a)
        pltpu.sync_copy(rows_a, out_hbm.at[pl.ds(c * NP + s * RPT + i * K, K)])
        return 0
    lax.fori_loop(0, RPT // K, copy_out, 0)


def _mm1_body(x_ref, w_ref, h_ref):
    h_ref[...] = jnp.dot(x_ref[...], w_ref[...],
                         preferred_element_type=jnp.float32)


_mm1 = pl.pallas_call(
    _mm1_body,
    out_shape=jax.ShapeDtypeStruct((N, D), jnp.float32),
)


def _scale_body(deg0_ref, deg1_ref, h_ref, hp_ref, dinv_ref):
    deg = deg0_ref[...][:, 0:1] + deg1_ref[...][:, 0:1] + 1.0
    dinv = lax.rsqrt(deg)
    hp_ref[...] = h_ref[...] * dinv
    dinv_ref[...] = jnp.broadcast_to(dinv, (N, D))


_scale = pl.pallas_call(
    _scale_body,
    out_shape=(jax.ShapeDtypeStruct((N, D), jnp.float32),
               jax.ShapeDtypeStruct((N, D), jnp.float32)),
)


def _combine_mm_body(s0_ref, s1_ref, hp_ref, dinv_ref, w_ref, b_ref, out_ref):
    agg = s0_ref[...] + s1_ref[...] + hp_ref[...]
    o1 = dinv_ref[...] * agg + b_ref[...]
    out_ref[...] = dinv_ref[...] * jnp.dot(
        o1, w_ref[...], preferred_element_type=jnp.float32)


_combine_mm = pl.pallas_call(
    _combine_mm_body,
    out_shape=jax.ShapeDtypeStruct((N, D), jnp.float32),
)


def _final_body(s0_ref, s1_ref, hp_ref, dinv_ref, b_ref, batch_ref, out_ref):
    agg = s0_ref[...] + s1_ref[...] + hp_ref[...]
    o2 = dinv_ref[...] * agg + b_ref[...]
    ids = batch_ref[...]
    gid = lax.broadcasted_iota(jnp.int32, (G, N), 0)
    p = (gid == ids).astype(jnp.float32)
    ssum = jnp.dot(p, o2, preferred_element_type=jnp.float32)
    cnt = jnp.sum(p, axis=1, keepdims=True)
    out_ref[...] = ssum / jnp.maximum(cnt, 1.0)


_final = pl.pallas_call(
    _final_body,
    out_shape=jax.ShapeDtypeStruct((G, D), jnp.float32),
)


def kernel(x, edge_index, batch, W1, b1, W2, b2):
    pad = EP - E
    src = jnp.concatenate([edge_index[0], jnp.zeros((pad,), jnp.int32)])
    dst = jnp.concatenate([edge_index[1], jnp.full((pad,), N, jnp.int32)])

    h1 = _mm1(x, W1)
    degp = _deg_kernel(dst)
    deg0 = degp[:N]
    deg1 = degp[NP:NP + N]

    hp1, dinv = _scale(deg0, deg1, h1)
    s1 = _agg_kernel(hp1, src, dst)
    hp2 = _combine_mm(s1[:N], s1[NP:NP + N], hp1, dinv, W2,
                      b1.reshape(1, D))
    s2 = _agg_kernel(hp2, src, dst)
    return _final(s2[:N], s2[NP:NP + N], hp2, dinv, b2.reshape(1, D),
                  batch.reshape(1, N))


# speedup vs baseline: 1.1104x; 1.1104x over previous
import functools
import jax
import jax.numpy as jnp
from jax import lax
from jax.experimental import pallas as pl
from jax.experimental.pallas import tpu as pltpu
from jax.experimental.pallas import tpu_sc as plsc

N = 10000
D = 128
G = 16
E = 320000

NC = 2
NS = 16
NW = NC * NS
K = 128
CH = 80
CH2 = CH // 2
CH0 = 116
CH1 = 44
CHM2 = max(CH0, CH1) // 2
EPW = K * CH
EP = EPW * NW
NP = 10240
RPT = NP // NS

_mesh = plsc.VectorSubcoreMesh(core_axis_name="c", subcore_axis_name="s")


@functools.partial(
    pl.kernel,
    out_type=jax.ShapeDtypeStruct((NC * NP, D), jnp.float32),
    mesh=_mesh,
    scratch_types=[
        pltpu.VMEM((K,), jnp.int32),
        pltpu.VMEM((K, D), jnp.float32),
        pltpu.VMEM_SHARED((NP, D), jnp.float32),
    ],
)
def _deg_kernel(dst_hbm, out_hbm, dst_v, ones_v, acc_sh):
    c = lax.axis_index("c")
    s = lax.axis_index("s")
    wid = s * NC + c

    zero = jnp.zeros((16,), jnp.float32)

    def fill_zero_row(i, _):
        def fill_lane(j, _):
            ones_v[i, pl.ds(j * 16, 16)] = zero
            return 0
        lax.fori_loop(0, D // 16, fill_lane, 0)
        return 0
    lax.fori_loop(0, K, fill_zero_row, 0)

    def zero_acc(i, _):
        pltpu.sync_copy(ones_v, acc_sh.at[pl.ds(s * RPT + i * K, K)])
        return 0
    lax.fori_loop(0, RPT // K, zero_acc, 0)

    one = jnp.full((16,), 1.0, jnp.float32)

    def fill_one_row(i, _):
        def fill_lane(j, _):
            ones_v[i, pl.ds(j * 16, 16)] = one
            return 0
        lax.fori_loop(0, D // 16, fill_lane, 0)
        return 0
    lax.fori_loop(0, K, fill_one_row, 0)
    plsc.subcore_barrier()

    def body(i, _):
        base = pl.multiple_of(wid * EPW + i * K, 8)
        pltpu.sync_copy(dst_hbm.at[pl.ds(base, K)], dst_v)
        pltpu.sync_copy(ones_v, acc_sh.at[dst_v], add=True)
        return 0
    lax.fori_loop(0, CH, body, 0)

    plsc.subcore_barrier()

    def copy_out(i, _):
        pltpu.sync_copy(acc_sh.at[pl.ds(s * RPT + i * K, K)], ones_v)
        pltpu.sync_copy(ones_v, out_hbm.at[pl.ds(c * NP + s * RPT + i * K, K)])
        return 0
    lax.fori_loop(0, RPT // K, copy_out, 0)


@functools.partial(
    pl.kernel,
    out_type=jax.ShapeDtypeStruct((NC * NP, D), jnp.float32),
    mesh=_mesh,
    scratch_types=[
        pltpu.VMEM((K,), jnp.int32),
        pltpu.VMEM((K,), jnp.int32),
        pltpu.VMEM((K,), jnp.int32),
        pltpu.VMEM((K,), jnp.int32),
        pltpu.VMEM((K, D), jnp.float32),
        pltpu.VMEM((K, D), jnp.float32),
        pltpu.VMEM_SHARED((NP, D), jnp.float32),
        pltpu.SemaphoreType.DMA,
        pltpu.SemaphoreType.DMA,
    ],
)
def _agg_kernel(hp_hbm, src_hbm, dst_hbm, out_hbm,
                src_a, dst_a, src_b, dst_b, rows_a, rows_b, acc_sh, ga, gb):
    c = lax.axis_index("c")
    s = lax.axis_index("s")
    zero = jnp.zeros((16,), jnp.float32)

    def fill_zero_row(i, _):
        def fill_lane(j, _):
            rows_a[i, pl.ds(j * 16, 16)] = zero
            return 0
        lax.fori_loop(0, D // 16, fill_lane, 0)
        return 0
    lax.fori_loop(0, K, fill_zero_row, 0)

    def zero_acc(i, _):
        pltpu.sync_copy(rows_a, acc_sh.at[pl.ds(s * RPT + i * K, K)])
        return 0
    lax.fori_loop(0, RPT // K, zero_acc, 0)
    plsc.subcore_barrier()

    n2 = jnp.where(c == 0, CH0 // 2, CH1 // 2)
    cbase = jnp.where(c == 0, s * CH0, 16 * CH0 + s * CH1)

    pltpu.sync_copy(src_hbm.at[pl.ds(cbase * K, K)], src_a)
    pltpu.sync_copy(dst_hbm.at[pl.ds(cbase * K, K)], dst_a)
    pltpu.async_copy(hp_hbm.at[src_a], rows_a, ga)

    def body(j, _):
        @pl.when(j < n2)
        def _():
            i2 = pl.multiple_of((cbase + 2 * j + 1) * K, 8)
            pltpu.sync_copy(src_hbm.at[pl.ds(i2, K)], src_b)
            pltpu.sync_copy(dst_hbm.at[pl.ds(i2, K)], dst_b)
            pltpu.async_copy(hp_hbm.at[src_b], rows_b, gb)
            pltpu.make_async_copy(hp_hbm.at[src_a], rows_a, ga).wait()
            pltpu.sync_copy(rows_a, acc_sh.at[dst_a], add=True)

            @pl.when(j < n2 - 1)
            def _():
                i3 = pl.multiple_of((cbase + 2 * j + 2) * K, 8)
                pltpu.sync_copy(src_hbm.at[pl.ds(i3, K)], src_a)
                pltpu.sync_copy(dst_hbm.at[pl.ds(i3, K)], dst_a)
                pltpu.async_copy(hp_hbm.at[src_a], rows_a, ga)
            pltpu.make_async_copy(hp_hbm.at[src_b], rows_b, gb).wait()
            pltpu.sync_copy(rows_b, acc_sh.at[dst_b], add=True)
        return 0
    lax.fori_loop(0, CHM2, body, 0)

    plsc.subcore_barrier()

    def copy_out(i, _):
        pltpu.sync_copy(acc_sh.at[pl.ds(s * RPT + i * K, K)], rows_a)
        pltpu.sync_copy(rows_a, out_hbm.at[pl.ds(c * NP + s * RPT + i * K, K)])
        return 0
    lax.fori_loop(0, RPT // K, copy_out, 0)


def _mm1_body(x_ref, w_ref, h_ref):
    h_ref[...] = jnp.dot(x_ref[...], w_ref[...],
                         preferred_element_type=jnp.float32)


_mm1 = pl.pallas_call(
    _mm1_body,
    out_shape=jax.ShapeDtypeStruct((N, D), jnp.float32),
)


def _scale_body(deg0_ref, deg1_ref, h_ref, hp_ref, dinv_ref):
    deg = deg0_ref[...][:, 0:1] + deg1_ref[...][:, 0:1] + 1.0
    dinv = lax.rsqrt(deg)
    hp_ref[...] = h_ref[...] * dinv
    dinv_ref[...] = jnp.broadcast_to(dinv, (N, D))


_scale = pl.pallas_call(
    _scale_body,
    out_shape=(jax.ShapeDtypeStruct((N, D), jnp.float32),
               jax.ShapeDtypeStruct((N, D), jnp.float32)),
)


def _combine_mm_body(s0_ref, s1_ref, hp_ref, dinv_ref, w_ref, b_ref, out_ref):
    agg = s0_ref[...] + s1_ref[...] + hp_ref[...]
    o1 = dinv_ref[...] * agg + b_ref[...]
    out_ref[...] = dinv_ref[...] * jnp.dot(
        o1, w_ref[...], preferred_element_type=jnp.float32)


_combine_mm = pl.pallas_call(
    _combine_mm_body,
    out_shape=jax.ShapeDtypeStruct((N, D), jnp.float32),
)


def _final_body(s0_ref, s1_ref, hp_ref, dinv_ref, b_ref, batch_ref, out_ref):
    agg = s0_ref[...] + s1_ref[...] + hp_ref[...]
    o2 = dinv_ref[...] * agg + b_ref[...]
    ids = batch_ref[...]
    gid = lax.broadcasted_iota(jnp.int32, (G, N), 0)
    p = (gid == ids).astype(jnp.float32)
    ssum = jnp.dot(p, o2, preferred_element_type=jnp.float32)
    cnt = jnp.sum(p, axis=1, keepdims=True)
    out_ref[...] = ssum / jnp.maximum(cnt, 1.0)


_final = pl.pallas_call(
    _final_body,
    out_shape=jax.ShapeDtypeStruct((G, D), jnp.float32),
)


def kernel(x, edge_index, batch, W1, b1, W2, b2):
    pad = EP - E
    src = jnp.concatenate([edge_index[0], jnp.zeros((pad,), jnp.int32)])
    dst = jnp.concatenate([edge_index[1], jnp.full((pad,), N, jnp.int32)])

    h1 = _mm1(x, W1)
    degp = _deg_kernel(dst)
    deg0 = degp[:N]
    deg1 = degp[NP:NP + N]

    hp1, dinv = _scale(deg0, deg1, h1)
    s1 = _agg_kernel(hp1, src, dst)
    hp2 = _combine_mm(s1[:N], s1[NP:NP + N], hp1, dinv, W2,
                      b1.reshape(1, D))
    s2 = _agg_kernel(hp2, src, dst)
    return _final(s2[:N], s2[NP:NP + N], hp2, dinv, b2.reshape(1, D),
                  batch.reshape(1, N))


# speedup vs baseline: 1.1122x; 1.0016x over previous
import functools
import jax
import jax.numpy as jnp
from jax import lax
from jax.experimental import pallas as pl
from jax.experimental.pallas import tpu as pltpu
from jax.experimental.pallas import tpu_sc as plsc

N = 10000
D = 128
G = 16
E = 320000

NC = 2
NS = 16
NW = NC * NS
K = 128
CH = 80
CH2 = CH // 2
CH0 = 132
CH1 = 28
CHM2 = max(CH0, CH1) // 2
EPW = K * CH
EP = EPW * NW
NP = 10240
RPT = NP // NS

_mesh = plsc.VectorSubcoreMesh(core_axis_name="c", subcore_axis_name="s")


@functools.partial(
    pl.kernel,
    out_type=jax.ShapeDtypeStruct((NC * NP, D), jnp.float32),
    mesh=_mesh,
    scratch_types=[
        pltpu.VMEM((K,), jnp.int32),
        pltpu.VMEM((K, D), jnp.float32),
        pltpu.VMEM_SHARED((NP, D), jnp.float32),
    ],
)
def _deg_kernel(dst_hbm, out_hbm, dst_v, ones_v, acc_sh):
    c = lax.axis_index("c")
    s = lax.axis_index("s")
    wid = s * NC + c

    zero = jnp.zeros((16,), jnp.float32)

    def fill_zero_row(i, _):
        def fill_lane(j, _):
            ones_v[i, pl.ds(j * 16, 16)] = zero
            return 0
        lax.fori_loop(0, D // 16, fill_lane, 0)
        return 0
    lax.fori_loop(0, K, fill_zero_row, 0)

    def zero_acc(i, _):
        pltpu.sync_copy(ones_v, acc_sh.at[pl.ds(s * RPT + i * K, K)])
        return 0
    lax.fori_loop(0, RPT // K, zero_acc, 0)

    one = jnp.full((16,), 1.0, jnp.float32)

    def fill_one_row(i, _):
        def fill_lane(j, _):
            ones_v[i, pl.ds(j * 16, 16)] = one
            return 0
        lax.fori_loop(0, D // 16, fill_lane, 0)
        return 0
    lax.fori_loop(0, K, fill_one_row, 0)
    plsc.subcore_barrier()

    def body(i, _):
        base = pl.multiple_of(wid * EPW + i * K, 8)
        pltpu.sync_copy(dst_hbm.at[pl.ds(base, K)], dst_v)
        pltpu.sync_copy(ones_v, acc_sh.at[dst_v], add=True)
        return 0
    lax.fori_loop(0, CH, body, 0)

    plsc.subcore_barrier()

    def copy_out(i, _):
        pltpu.sync_copy(acc_sh.at[pl.ds(s * RPT + i * K, K)], ones_v)
        pltpu.sync_copy(ones_v, out_hbm.at[pl.ds(c * NP + s * RPT + i * K, K)])
        return 0
    lax.fori_loop(0, RPT // K, copy_out, 0)


@functools.partial(
    pl.kernel,
    out_type=jax.ShapeDtypeStruct((NC * NP, D), jnp.float32),
    mesh=_mesh,
    scratch_types=[
        pltpu.VMEM((K,), jnp.int32),
        pltpu.VMEM((K,), jnp.int32),
        pltpu.VMEM((K,), jnp.int32),
        pltpu.VMEM((K,), jnp.int32),
        pltpu.VMEM((K, D), jnp.float32),
        pltpu.VMEM((K, D), jnp.float32),
        pltpu.VMEM_SHARED((NP, D), jnp.float32),
        pltpu.SemaphoreType.DMA,
        pltpu.SemaphoreType.DMA,
    ],
)
def _agg_kernel(hp_hbm, src_hbm, dst_hbm, out_hbm,
                src_a, dst_a, src_b, dst_b, rows_a, rows_b, acc_sh, ga, gb):
    c = lax.axis_index("c")
    s = lax.axis_index("s")
    zero = jnp.zeros((16,), jnp.float32)

    def fill_zero_row(i, _):
        def fill_lane(j, _):
            rows_a[i, pl.ds(j * 16, 16)] = zero
            return 0
        lax.fori_loop(0, D // 16, fill_lane, 0)
        return 0
    lax.fori_loop(0, K, fill_zero_row, 0)

    def zero_acc(i, _):
        pltpu.sync_copy(rows_a, acc_sh.at[pl.ds(s * RPT + i * K, K)])
        return 0
    lax.fori_loop(0, RPT // K, zero_acc, 0)
    plsc.subcore_barrier()

    n2 = jnp.where(c == 0, CH0 // 2, CH1 // 2)
    cbase = jnp.where(c == 0, s * CH0, 16 * CH0 + s * CH1)

    pltpu.sync_copy(src_hbm.at[pl.ds(cbase * K, K)], src_a)
    pltpu.sync_copy(dst_hbm.at[pl.ds(cbase * K, K)], dst_a)
    pltpu.async_copy(hp_hbm.at[src_a], rows_a, ga)

    def body(j, _):
        @pl.when(j < n2)
        def _():
            i2 = pl.multiple_of((cbase + 2 * j + 1) * K, 8)
            pltpu.sync_copy(src_hbm.at[pl.ds(i2, K)], src_b)
            pltpu.sync_copy(dst_hbm.at[pl.ds(i2, K)], dst_b)
            pltpu.async_copy(hp_hbm.at[src_b], rows_b, gb)
            pltpu.make_async_copy(hp_hbm.at[src_a], rows_a, ga).wait()
            pltpu.sync_copy(rows_a, acc_sh.at[dst_a], add=True)

            @pl.when(j < n2 - 1)
            def _():
                i3 = pl.multiple_of((cbase + 2 * j + 2) * K, 8)
                pltpu.sync_copy(src_hbm.at[pl.ds(i3, K)], src_a)
                pltpu.sync_copy(dst_hbm.at[pl.ds(i3, K)], dst_a)
                pltpu.async_copy(hp_hbm.at[src_a], rows_a, ga)
            pltpu.make_async_copy(hp_hbm.at[src_b], rows_b, gb).wait()
            pltpu.sync_copy(rows_b, acc_sh.at[dst_b], add=True)
        return 0
    lax.fori_loop(0, CHM2, body, 0)

    plsc.subcore_barrier()

    def copy_out(i, _):
        pltpu.sync_copy(acc_sh.at[pl.ds(s * RPT + i * K, K)], rows_a)
        pltpu.sync_copy(rows_a, out_hbm.at[pl.ds(c * NP + s * RPT + i * K, K)])
        return 0
    lax.fori_loop(0, RPT // K, copy_out, 0)


def _mm1_body(x_ref, w_ref, h_ref):
    h_ref[...] = jnp.dot(x_ref[...], w_ref[...],
                         preferred_element_type=jnp.float32)


_mm1 = pl.pallas_call(
    _mm1_body,
    out_shape=jax.ShapeDtypeStruct((N, D), jnp.float32),
)


def _scale_body(deg0_ref, deg1_ref, h_ref, hp_ref, dinv_ref):
    deg = deg0_ref[...][:, 0:1] + deg1_ref[...][:, 0:1] + 1.0
    dinv = lax.rsqrt(deg)
    hp_ref[...] = h_ref[...] * dinv
    dinv_ref[...] = jnp.broadcast_to(dinv, (N, D))


_scale = pl.pallas_call(
    _scale_body,
    out_shape=(jax.ShapeDtypeStruct((N, D), jnp.float32),
               jax.ShapeDtypeStruct((N, D), jnp.float32)),
)


def _combine_mm_body(s0_ref, s1_ref, hp_ref, dinv_ref, w_ref, b_ref, out_ref):
    agg = s0_ref[...] + s1_ref[...] + hp_ref[...]
    o1 = dinv_ref[...] * agg + b_ref[...]
    out_ref[...] = dinv_ref[...] * jnp.dot(
        o1, w_ref[...], preferred_element_type=jnp.float32)


_combine_mm = pl.pallas_call(
    _combine_mm_body,
    out_shape=jax.ShapeDtypeStruct((N, D), jnp.float32),
)


def _final_body(s0_ref, s1_ref, hp_ref, dinv_ref, b_ref, batch_ref, out_ref):
    agg = s0_ref[...] + s1_ref[...] + hp_ref[...]
    o2 = dinv_ref[...] * agg + b_ref[...]
    ids = batch_ref[...]
    gid = lax.broadcasted_iota(jnp.int32, (G, N), 0)
    p = (gid == ids).astype(jnp.float32)
    ssum = jnp.dot(p, o2, preferred_element_type=jnp.float32)
    cnt = jnp.sum(p, axis=1, keepdims=True)
    out_ref[...] = ssum / jnp.maximum(cnt, 1.0)


_final = pl.pallas_call(
    _final_body,
    out_shape=jax.ShapeDtypeStruct((G, D), jnp.float32),
)


def kernel(x, edge_index, batch, W1, b1, W2, b2):
    pad = EP - E
    src = jnp.concatenate([edge_index[0], jnp.zeros((pad,), jnp.int32)])
    dst = jnp.concatenate([edge_index[1], jnp.full((pad,), N, jnp.int32)])

    h1 = _mm1(x, W1)
    degp = _deg_kernel(dst)
    deg0 = degp[:N]
    deg1 = degp[NP:NP + N]

    hp1, dinv = _scale(deg0, deg1, h1)
    s1 = _agg_kernel(hp1, src, dst)
    hp2 = _combine_mm(s1[:N], s1[NP:NP + N], hp1, dinv, W2,
                      b1.reshape(1, D))
    s2 = _agg_kernel(hp2, src, dst)
    return _final(s2[:N], s2[NP:NP + N], hp2, dinv, b2.reshape(1, D),
                  batch.reshape(1, N))
